# baseline (speedup 1.0000x reference)
import functools

import jax
import jax.numpy as jnp
from jax import lax
from jax.experimental import pallas as pl
from jax.experimental.pallas import tpu as pltpu

try:
    from jax.experimental.pallas import tpu_sc as plsc
    _HAS_SC = True
except ImportError:
    plsc = None
    _HAS_SC = False

N = 10000
K = 20
NP = 10240
BR = 256
GRID = NP // BR


SUB = 32
NSUB = BR // SUB
TILES = NP // 128
BIGI = 2 ** 30
MINF = -jnp.inf


def _topk_kernel(xt_ref, xall_ref, idx_ref, d_ref):
    xt = xt_ref[...]
    xall = xall_ref[...]
    inner = -2.0 * jnp.dot(xt, xall, preferred_element_type=jnp.float32)
    xx_cols = jnp.sum(xall * xall, axis=0, keepdims=True)
    xx_rows = jnp.sum(xt * xt, axis=1, keepdims=True)
    d = -xx_cols - inner - xx_rows
    colf = lax.broadcasted_iota(jnp.int32, (BR, NP), 1)
    d_ref[...] = jnp.where(colf < N, d, MINF)

    lane = lax.broadcasted_iota(jnp.int32, (SUB, 128), 1)

    for sub in range(NSUB):
        r0 = sub * SUB
        finit = jnp.full((SUB, 128), MINF, jnp.float32)
        ginit = jnp.full((SUB, 128), BIGI, jnp.int32)

        def fold(t, c):
            f1, g1, f2, g2, f3, g3, f4, g4 = c
            key = d_ref[pl.ds(r0, SUB), pl.ds(t * 128, 128)]
            g = lane + t * 128
            gt1 = key > f1
            gt2 = key > f2
            gt3 = key > f3
            gt4 = key > f4
            nf4 = jnp.where(gt3, f3, jnp.where(gt4, key, f4))
            ng4 = jnp.where(gt3, g3, jnp.where(gt4, g, g4))
            nf3 = jnp.where(gt2, f2, jnp.where(gt3, key, f3))
            ng3 = jnp.where(gt2, g2, jnp.where(gt3, g, g3))
            nf2 = jnp.where(gt1, f1, jnp.where(gt2, key, f2))
            ng2 = jnp.where(gt1, g1, jnp.where(gt2, g, g2))
            nf1 = jnp.where(gt1, key, f1)
            ng1 = jnp.where(gt1, g, g1)
            return nf1, ng1, nf2, ng2, nf3, ng3, nf4, ng4

        f1, g1, f2, g2, f3, g3, f4, g4 = lax.fori_loop(
            0, TILES, fold,
            (finit, ginit, finit, ginit, finit, ginit, finit, ginit),
            unroll=16)

        acc0 = jnp.zeros((SUB, 128), jnp.int32)
        bound0 = jnp.full((SUB, 1), MINF, jnp.float32)
        valid0 = jnp.ones((SUB, 1), jnp.int32)

        def extract(j, c):
            f1, g1, f2, g2, f3, g3, f4, g4, acc, bound, valid = c
            m = jnp.max(f1, axis=1, keepdims=True)
            cand = jnp.where(f1 == m, g1, BIGI)
            gstar = jnp.min(cand, axis=1, keepdims=True)
            valid = jnp.where(m > bound, valid, 0)
            cl = (f1 == m) & (g1 == gstar)
            row_empty = jnp.any(cl & (f2 == MINF), axis=1, keepdims=True)
            bound = jnp.where(row_empty, jnp.maximum(bound, m), bound)
            acc = jnp.where(lane == j, gstar, acc)
            f1 = jnp.where(cl, f2, f1)
            g1 = jnp.where(cl, g2, g1)
            f2 = jnp.where(cl, f3, f2)
            g2 = jnp.where(cl, g3, g2)
            f3 = jnp.where(cl, f4, f3)
            g3 = jnp.where(cl, g4, g3)
            f4 = jnp.where(cl, MINF, f4)
            g4 = jnp.where(cl, BIGI, g4)
            return f1, g1, f2, g2, f3, g3, f4, g4, acc, bound, valid

        out = lax.fori_loop(
            0, K, extract,
            (f1, g1, f2, g2, f3, g3, f4, g4, acc0, bound0, valid0),
            unroll=4)
        acc, valid = out[8], out[10]
        idx_ref[pl.ds(r0, SUB), :] = acc[:, :K]

        @pl.when(jnp.min(valid) == 0)
        def _():
            col = lax.broadcasted_iota(jnp.int32, (SUB, NP), 1)

            def slow(t, acc):
                dsub = d_ref[pl.ds(r0, SUB), :]
                mm = jnp.max(dsub, axis=1, keepdims=True)
                cnd = jnp.where(dsub >= mm, col, NP)
                it = jnp.min(cnd, axis=1, keepdims=True)
                acc = jnp.where(lane == t, it, acc)
                d_ref[pl.ds(r0, SUB), :] = jnp.where(col == it, MINF, dsub)
                return acc

            acc2 = lax.fori_loop(0, K, slow, jnp.zeros((SUB, 128), jnp.int32))
            idx_ref[pl.ds(r0, SUB), :] = acc2[:, :K]


def _knn_topk(coor2d):
    xall = jnp.pad(coor2d, ((0, 0), (0, NP - N)))
    xt = xall.T
    return pl.pallas_call(
        _topk_kernel,
        grid=(GRID,),
        in_specs=[
            pl.BlockSpec((BR, 3), lambda i: (i, 0)),
            pl.BlockSpec((3, NP), lambda i: (0, 0)),
        ],
        out_specs=pl.BlockSpec((BR, K), lambda i: (i, 0)),
        out_shape=jax.ShapeDtypeStruct((NP, K), jnp.int32),
        scratch_shapes=[pltpu.VMEM((BR, NP), jnp.float32)],
    )(xt, xall)[:N]


def _make_sc_gather(nw, chunks, d):
    b_per_w = chunks * 128
    bp = nw * b_per_w
    info = plsc.get_sparse_core_info()
    mesh = plsc.VectorSubcoreMesh(core_axis_name="c", subcore_axis_name="s")

    @functools.partial(
        pl.kernel,
        mesh=mesh,
        out_type=jax.ShapeDtypeStruct((bp, d), jnp.float32),
        scratch_types=[
            pltpu.VMEM((chunks, 128), jnp.int32),
            pltpu.VMEM((b_per_w, d), jnp.float32),
            pltpu.SemaphoreType.DMA,
        ],
        compiler_params=pltpu.CompilerParams(use_tc_tiling_on_sc=False),
    )
    def gather_k(table_hbm, idx_hbm, out_hbm, idx_v, rows_v, sem):
        wid = lax.axis_index("s") * info.num_cores + lax.axis_index("c")
        base = wid * b_per_w
        pltpu.sync_copy(idx_hbm.at[wid], idx_v)

        def fire(j):
            pltpu.async_copy(
                table_hbm.at[idx_v.at[j]],
                rows_v.at[pl.ds(j * 128, 128)],
                sem,
            )

        pl.loop(0, chunks)(fire)
        pltpu.make_async_copy(
            table_hbm.at[pl.ds(0, b_per_w)], rows_v, sem).wait()
        pltpu.sync_copy(rows_v, out_hbm.at[pl.ds(base, b_per_w)])

    return gather_k


def kernel(coor, nor):
    B, C, n = coor.shape
    cn = nor.shape[1]
    coor2d = coor[0]
    idx = _knn_topk(coor2d)

    coor_t = coor2d.T
    nor_t = nor[0].T
    table = jnp.concatenate(
        [coor_t, nor_t, jnp.zeros((n, 16 - C - cn), jnp.float32)], axis=1)

    idx_flat = idx.reshape(-1)
    nw = 32
    chunks = -(-idx_flat.shape[0] // (128 * nw))
    chunks = -(-chunks // 8) * 8
    bp = nw * chunks * 128
    idx_pad = jnp.pad(idx_flat, (0, bp - idx_flat.shape[0]))
    idx_3d = idx_pad.reshape(nw, chunks, 128)

    gathered = _make_sc_gather(nw, chunks, 16)(table, idx_3d)[: n * K]
    g = gathered.reshape(B, n, K, 16)

    coor_feature = jnp.concatenate(
        [g[..., :C], jnp.broadcast_to(coor_t.reshape(B, n, 1, C), (B, n, K, C))],
        axis=3)
    coor_feature = jnp.transpose(coor_feature, (0, 3, 1, 2))
    nor_feature = jnp.concatenate(
        [g[..., C:C + cn],
         jnp.broadcast_to(nor_t.reshape(B, n, 1, cn), (B, n, K, cn))],
        axis=3)
    nor_feature = jnp.transpose(nor_feature, (0, 3, 1, 2))
    return coor_feature, nor_feature, idx.reshape(B, n, K)


# speedup vs baseline: 1.0588x; 1.0588x over previous
import functools

import jax
import jax.numpy as jnp
from jax import lax
from jax.experimental import pallas as pl
from jax.experimental.pallas import tpu as pltpu

try:
    from jax.experimental.pallas import tpu_sc as plsc
    _HAS_SC = True
except ImportError:
    plsc = None
    _HAS_SC = False

N = 10000
K = 20
NP = 10240
BR = 256
GRID = NP // BR


SUB = 32
NSUB = BR // SUB
TILES = NP // 128
BIGI = 2 ** 30
MINF = -jnp.inf


def _topk_kernel(xt_ref, xall_ref, idx_ref, d_ref):
    xt = xt_ref[...]
    xall = xall_ref[...]
    inner = -2.0 * jnp.dot(xt, xall, preferred_element_type=jnp.float32)
    xx_cols = jnp.sum(xall * xall, axis=0, keepdims=True)
    xx_rows = jnp.sum(xt * xt, axis=1, keepdims=True)
    d = -xx_cols - inner - xx_rows
    colf = lax.broadcasted_iota(jnp.int32, (BR, NP), 1)
    d_ref[...] = jnp.where(colf < N, d, MINF)

    lane = lax.broadcasted_iota(jnp.int32, (SUB, 128), 1)

    for sub in range(NSUB):
        r0 = sub * SUB
        finit = jnp.full((SUB, 128), MINF, jnp.float32)
        ginit = jnp.full((SUB, 128), BIGI, jnp.int32)

        def fold(t, c):
            f1, g1, f2, g2, f3, g3, f4, g4 = c
            key = d_ref[pl.ds(r0, SUB), pl.ds(t * 128, 128)]
            g = lane + t * 128
            gt1 = key > f1
            gt2 = key > f2
            gt3 = key > f3
            gt4 = key > f4
            nf4 = jnp.where(gt3, f3, jnp.where(gt4, key, f4))
            ng4 = jnp.where(gt3, g3, jnp.where(gt4, g, g4))
            nf3 = jnp.where(gt2, f2, jnp.where(gt3, key, f3))
            ng3 = jnp.where(gt2, g2, jnp.where(gt3, g, g3))
            nf2 = jnp.where(gt1, f1, jnp.where(gt2, key, f2))
            ng2 = jnp.where(gt1, g1, jnp.where(gt2, g, g2))
            nf1 = jnp.where(gt1, key, f1)
            ng1 = jnp.where(gt1, g, g1)
            return nf1, ng1, nf2, ng2, nf3, ng3, nf4, ng4

        f1, g1, f2, g2, f3, g3, f4, g4 = lax.fori_loop(
            0, TILES, fold,
            (finit, ginit, finit, ginit, finit, ginit, finit, ginit),
            unroll=16)

        acc0 = jnp.zeros((SUB, 128), jnp.int32)
        bound0 = jnp.full((SUB, 1), MINF, jnp.float32)
        valid0 = jnp.ones((SUB, 1), jnp.int32)

        def extract(j, c):
            f1, g1, f2, g2, f3, g3, f4, g4, acc, bound, valid = c
            m = jnp.max(f1, axis=1, keepdims=True)
            cand = jnp.where(f1 == m, g1, BIGI)
            gstar = jnp.min(cand, axis=1, keepdims=True)
            valid = jnp.where(m > bound, valid, 0)
            cl = (f1 == m) & (g1 == gstar)
            row_empty = jnp.any(cl & (f2 == MINF), axis=1, keepdims=True)
            bound = jnp.where(row_empty, jnp.maximum(bound, m), bound)
            acc = jnp.where(lane == j, gstar, acc)
            f1 = jnp.where(cl, f2, f1)
            g1 = jnp.where(cl, g2, g1)
            f2 = jnp.where(cl, f3, f2)
            g2 = jnp.where(cl, g3, g2)
            f3 = jnp.where(cl, f4, f3)
            g3 = jnp.where(cl, g4, g3)
            f4 = jnp.where(cl, MINF, f4)
            g4 = jnp.where(cl, BIGI, g4)
            return f1, g1, f2, g2, f3, g3, f4, g4, acc, bound, valid

        out = lax.fori_loop(
            0, K, extract,
            (f1, g1, f2, g2, f3, g3, f4, g4, acc0, bound0, valid0),
            unroll=True)
        acc, valid = out[8], out[10]
        idx_ref[pl.ds(r0, SUB), :] = acc[:, :K]

        @pl.when(jnp.min(valid) == 0)
        def _():
            col = lax.broadcasted_iota(jnp.int32, (SUB, NP), 1)

            def slow(t, acc):
                dsub = d_ref[pl.ds(r0, SUB), :]
                mm = jnp.max(dsub, axis=1, keepdims=True)
                cnd = jnp.where(dsub >= mm, col, NP)
                it = jnp.min(cnd, axis=1, keepdims=True)
                acc = jnp.where(lane == t, it, acc)
                d_ref[pl.ds(r0, SUB), :] = jnp.where(col == it, MINF, dsub)
                return acc

            acc2 = lax.fori_loop(0, K, slow, jnp.zeros((SUB, 128), jnp.int32))
            idx_ref[pl.ds(r0, SUB), :] = acc2[:, :K]


def _knn_topk(coor2d):
    xall = jnp.pad(coor2d, ((0, 0), (0, NP - N)))
    xt = xall.T
    return pl.pallas_call(
        _topk_kernel,
        grid=(GRID,),
        in_specs=[
            pl.BlockSpec((BR, 3), lambda i: (i, 0)),
            pl.BlockSpec((3, NP), lambda i: (0, 0)),
        ],
        out_specs=pl.BlockSpec((BR, K), lambda i: (i, 0)),
        out_shape=jax.ShapeDtypeStruct((NP, K), jnp.int32),
        scratch_shapes=[pltpu.VMEM((BR, NP), jnp.float32)],
    )(xt, xall)[:N]


def _make_sc_gather(nw, chunks, d):
    b_per_w = chunks * 128
    bp = nw * b_per_w
    info = plsc.get_sparse_core_info()
    mesh = plsc.VectorSubcoreMesh(core_axis_name="c", subcore_axis_name="s")

    @functools.partial(
        pl.kernel,
        mesh=mesh,
        out_type=jax.ShapeDtypeStruct((bp, d), jnp.float32),
        scratch_types=[
            pltpu.VMEM((chunks, 128), jnp.int32),
            pltpu.VMEM((b_per_w, d), jnp.float32),
            pltpu.SemaphoreType.DMA,
        ],
        compiler_params=pltpu.CompilerParams(use_tc_tiling_on_sc=False),
    )
    def gather_k(table_hbm, idx_hbm, out_hbm, idx_v, rows_v, sem):
        wid = lax.axis_index("s") * info.num_cores + lax.axis_index("c")
        base = wid * b_per_w
        pltpu.sync_copy(idx_hbm.at[wid], idx_v)

        def fire(j):
            pltpu.async_copy(
                table_hbm.at[idx_v.at[j]],
                rows_v.at[pl.ds(j * 128, 128)],
                sem,
            )

        pl.loop(0, chunks)(fire)
        pltpu.make_async_copy(
            table_hbm.at[pl.ds(0, b_per_w)], rows_v, sem).wait()
        pltpu.sync_copy(rows_v, out_hbm.at[pl.ds(base, b_per_w)])

    return gather_k


def kernel(coor, nor):
    B, C, n = coor.shape
    cn = nor.shape[1]
    coor2d = coor[0]
    idx = _knn_topk(coor2d)

    coor_t = coor2d.T
    nor_t = nor[0].T
    table = jnp.concatenate(
        [coor_t, nor_t, jnp.zeros((n, 16 - C - cn), jnp.float32)], axis=1)

    idx_flat = idx.reshape(-1)
    nw = 32
    chunks = -(-idx_flat.shape[0] // (128 * nw))
    chunks = -(-chunks // 8) * 8
    bp = nw * chunks * 128
    idx_pad = jnp.pad(idx_flat, (0, bp - idx_flat.shape[0]))
    idx_3d = idx_pad.reshape(nw, chunks, 128)

    gathered = _make_sc_gather(nw, chunks, 16)(table, idx_3d)[: n * K]
    g = gathered.reshape(B, n, K, 16)

    coor_feature = jnp.concatenate(
        [g[..., :C], jnp.broadcast_to(coor_t.reshape(B, n, 1, C), (B, n, K, C))],
        axis=3)
    coor_feature = jnp.transpose(coor_feature, (0, 3, 1, 2))
    nor_feature = jnp.concatenate(
        [g[..., C:C + cn],
         jnp.broadcast_to(nor_t.reshape(B, n, 1, cn), (B, n, K, cn))],
        axis=3)
    nor_feature = jnp.transpose(nor_feature, (0, 3, 1, 2))
    return coor_feature, nor_feature, idx.reshape(B, n, K)
